# baseline (speedup 1.0000x reference)
import functools

import jax
import jax.numpy as jnp
from jax import lax
from jax.experimental import pallas as pl
from jax.experimental.pallas import tpu as pltpu
from jax.experimental.pallas import tpu_sc as plsc

N = 4096
R = 5
NB = 30
H0 = 256
H1 = 64
E = 131072
U = 2048

NC, NS, L = 2, 16, 16
NT = NC * NS
EPT = E // NT


def _projfc_body(a_ref, fc_ref, out_ref):
    out_ref[...] = jnp.dot(a_ref[...], fc_ref[...],
                           preferred_element_type=jnp.float32)


def _projfc(a2d, fc_w):
    bm = 2048
    m = a2d.shape[0]
    return pl.pallas_call(
        _projfc_body,
        grid=(m // bm,),
        in_specs=[
            pl.BlockSpec((bm, H0), lambda i: (i, 0)),
            pl.BlockSpec((H0, H1), lambda i: (0, 0)),
        ],
        out_specs=pl.BlockSpec((bm, H1), lambda i: (i, 0)),
        out_shape=jax.ShapeDtypeStruct((m, H1), jnp.float32),
    )(a2d, fc_w)


def _wf_body(comp_ref, bf_ref, rootf_ref, out_ref):
    w = jnp.dot(comp_ref[...], bf_ref[...],
                preferred_element_type=jnp.float32)
    out_ref[...] = jnp.concatenate([w, rootf_ref[...]], axis=0)


def _wf(comp, bf_flat, rootf_flat):
    cb = 8192
    return pl.pallas_call(
        _wf_body,
        grid=(bf_flat.shape[1] // cb,),
        in_specs=[
            pl.BlockSpec((R, NB), lambda c: (0, 0)),
            pl.BlockSpec((NB, cb), lambda c: (0, c)),
            pl.BlockSpec((1, cb), lambda c: (0, c)),
        ],
        out_specs=pl.BlockSpec((R + 1, cb), lambda c: (0, c)),
        out_shape=jax.ShapeDtypeStruct((R + 1, N * H1), jnp.float32),
    )(comp, bf_flat, rootf_flat)


def _xwf_body(x_ref, w_ref, out_ref):
    k = pl.program_id(1)
    bn = x_ref.shape[0]
    for r in range(R + 1):
        p = jnp.dot(x_ref[...], w_ref[r], preferred_element_type=jnp.float32)
        p = jnp.concatenate([p, jnp.zeros((bn, 128 - H1), jnp.float32)],
                            axis=1)

        @pl.when(k == 0)
        def _():
            out_ref[r] = p

        @pl.when(k > 0)
        def _():
            out_ref[r] = out_ref[r] + p


def _xwf(x, w6):
    bn, bk = 1024, 512
    return pl.pallas_call(
        _xwf_body,
        grid=(N // bn, N // bk),
        in_specs=[
            pl.BlockSpec((bn, bk), lambda n, k: (n, k)),
            pl.BlockSpec((R + 1, bk, H1), lambda n, k: (0, k, 0)),
        ],
        out_specs=pl.BlockSpec((R + 1, bn, 128), lambda n, k: (0, n, 0)),
        out_shape=jax.ShapeDtypeStruct((R + 1, N, 128), jnp.float32),
    )(x, w6)


def _sck1(edge_index, edge_type):
    ka = 1024
    nchunks = EPT // ka
    zc = R * N // NS
    mesh = plsc.VectorSubcoreMesh(core_axis_name="c", subcore_axis_name="s")

    @functools.partial(
        pl.kernel,
        out_type=jax.ShapeDtypeStruct((NC, R * N), jnp.float32),
        mesh=mesh,
        scratch_types=[
            pltpu.VMEM_SHARED((R * N,), jnp.float32),
            pltpu.VMEM((ka,), jnp.int32),
            pltpu.VMEM((ka,), jnp.int32),
            pltpu.VMEM((ka,), jnp.int32),
            pltpu.VMEM((ka,), jnp.float32),
            pltpu.VMEM((zc,), jnp.float32),
        ],
    )
    def k(ei, et, cnt_out, cnt_s, dbuf, tbuf, cidx, ones, zbuf):
        c = lax.axis_index("c")
        s = lax.axis_index("s")
        wid = c * NS + s

        def zf(i, _):
            zbuf[pl.ds(i * L, L)] = jnp.zeros((L,), jnp.float32)
            return 0

        lax.fori_loop(0, zc // L, zf, 0)

        def of(i, _):
            ones[pl.ds(i * L, L)] = jnp.ones((L,), jnp.float32)
            return 0

        lax.fori_loop(0, ka // L, of, 0)

        pltpu.sync_copy(zbuf, cnt_s.at[pl.ds(s * zc, zc)])
        plsc.subcore_barrier()

        def chunk(i, _):
            base = wid * EPT + i * ka
            pltpu.sync_copy(ei.at[1, pl.ds(base, ka)], dbuf)
            pltpu.sync_copy(et.at[pl.ds(base, ka)], tbuf)

            def vec(j, _):
                d = dbuf[pl.ds(j * L, L)]
                t = tbuf[pl.ds(j * L, L)]
                cidx[pl.ds(j * L, L)] = t * N + d
                return 0

            lax.fori_loop(0, ka // L, vec, 0)
            pltpu.sync_copy(ones, cnt_s.at[cidx], add=True)
            return 0

        lax.fori_loop(0, nchunks, chunk, 0)
        plsc.subcore_barrier()
        pltpu.sync_copy(cnt_s.at[pl.ds(s * zc, zc)],
                        cnt_out.at[c, pl.ds(s * zc, zc)])

    return k(edge_index, edge_type)


def _rcnt_body(cnt_ref, out_ref):
    out_ref[...] = 1.0 / jnp.maximum(cnt_ref[0] + cnt_ref[1], 1.0)


def _rcnt(cnt2r):
    return pl.pallas_call(
        _rcnt_body,
        grid=(1,),
        in_specs=[pl.BlockSpec((NC, R * N // 128, 128), lambda i: (0, 0, 0))],
        out_specs=pl.BlockSpec((R * N // 128, 128), lambda i: (0, 0)),
        out_shape=jax.ShapeDtypeStruct((R * N // 128, 128), jnp.float32),
    )(cnt2r)


def _sck2(xwf_flat, edge_index, edge_type, rcnt_flat):
    kb = 128
    nchunks = EPT // kb
    spt = N * H1 // NS
    mesh = plsc.VectorSubcoreMesh(core_axis_name="c", subcore_axis_name="s")

    @functools.partial(
        pl.kernel,
        out_type=jax.ShapeDtypeStruct((NC * N * H1,), jnp.float32),
        mesh=mesh,
        scratch_types=[
            pltpu.VMEM_SHARED((N * H1,), jnp.float32),
            pltpu.VMEM((kb,), jnp.int32),
            pltpu.VMEM((kb,), jnp.int32),
            pltpu.VMEM((kb,), jnp.int32),
            pltpu.VMEM((kb,), jnp.int32),
            pltpu.VMEM((kb,), jnp.float32),
            pltpu.VMEM((kb, 128), jnp.float32),
            pltpu.VMEM((kb * H1,), jnp.float32),
            pltpu.VMEM((kb * H1,), jnp.int32),
            pltpu.VMEM((kb * H1,), jnp.float32),
            pltpu.VMEM((kb * H1,), jnp.int32),
            pltpu.VMEM((2048,), jnp.float32),
            pltpu.SemaphoreType.DMA,
            pltpu.SemaphoreType.DMA,
            pltpu.SemaphoreType.DMA,
            pltpu.SemaphoreType.DMA,
        ],
    )
    def k(xwf, ei, et, rcnt, msg_out, acc_s, sbuf, dbuf, tbuf, gidx, wbuf,
          rows, vals_a, eidx_a, vals_b, eidx_b, zbuf, sem, sem2, sem3, sem4):
        c = lax.axis_index("c")
        s = lax.axis_index("s")
        wid = c * NS + s

        def zf(i, _):
            zbuf[pl.ds(i * L, L)] = jnp.zeros((L,), jnp.float32)
            return 0

        lax.fori_loop(0, 2048 // L, zf, 0)
        for jj in range(spt // 2048):
            pltpu.sync_copy(zbuf, acc_s.at[pl.ds(s * spt + jj * 2048, 2048)])
        plsc.subcore_barrier()

        iotas = [lax.iota(jnp.int32, L) + h * L for h in range(H1 // L)]

        def subchunk(i, vals, eidx):
            base = wid * EPT + i * kb
            pltpu.sync_copy(ei.at[0, pl.ds(base, kb)], sbuf)
            pltpu.sync_copy(ei.at[1, pl.ds(base, kb)], dbuf)
            pltpu.sync_copy(et.at[pl.ds(base, kb)], tbuf)

            def vec(j, _):
                sv = sbuf[pl.ds(j * L, L)]
                dv = dbuf[pl.ds(j * L, L)]
                tv = tbuf[pl.ds(j * L, L)]
                gidx[pl.ds(j * L, L)] = tv * N + sv
                cidx = tv * N + dv
                dbuf[pl.ds(j * L, L)] = dv * H1
                tbuf[pl.ds(j * L, L)] = cidx
                return 0

            lax.fori_loop(0, kb // L, vec, 0)
            cp1 = pltpu.async_copy(xwf.at[gidx], rows, sem)
            cp2 = pltpu.async_copy(rcnt.at[tbuf], wbuf, sem2)
            cp1.wait()
            cp2.wait()

            def scale(j, _):
                wv = wbuf[pl.ds(j * L, L)]
                d64 = dbuf[pl.ds(j * L, L)]
                for e16 in range(L):
                    e = j * L + e16
                    w = wv[e16]
                    db = d64[e16]
                    for h in range(H1 // L):
                        vals[pl.ds(e * H1 + h * L, L)] = (
                            rows[e, pl.ds(h * L, L)] * w)
                        eidx[pl.ds(e * H1 + h * L, L)] = iotas[h] + db
                return 0

            lax.fori_loop(0, kb // L, scale, 0)

        def chunk(i, _):
            subchunk(2 * i, vals_a, eidx_a)
            cpa = pltpu.async_copy(vals_a, acc_s.at[eidx_a], sem3, add=True)
            subchunk(2 * i + 1, vals_b, eidx_b)
            cpa.wait()
            cpb = pltpu.async_copy(vals_b, acc_s.at[eidx_b], sem4, add=True)
            cpb.wait()
            return 0

        lax.fori_loop(0, nchunks // 2, chunk, 0)
        plsc.subcore_barrier()
        pltpu.sync_copy(acc_s.at[pl.ds(s * spt, spt)],
                        msg_out.at[pl.ds(c * N * H1 + s * spt, spt)])

    return k(xwf_flat, edge_index, edge_type, rcnt_flat)


def _relu_body(xwf_ref, msg_ref, bias_ref, fc_ref, h_ref, ht_ref):
    biasf = jnp.dot(bias_ref[...], fc_ref[...],
                    preferred_element_type=jnp.float32)
    h = jnp.maximum(xwf_ref[0, :, :H1] + msg_ref[0] + msg_ref[1] + biasf, 0.0)
    h_ref[...] = h
    ht_ref[...] = h.T


def _relu(xwf, msgf, bias2, fc_w):
    bn = 512
    return pl.pallas_call(
        _relu_body,
        grid=(N // bn,),
        in_specs=[
            pl.BlockSpec((1, bn, 128), lambda n: (R, n, 0)),
            pl.BlockSpec((NC, bn, H1), lambda n: (0, n, 0)),
            pl.BlockSpec((1, H0), lambda n: (0, 0)),
            pl.BlockSpec((H0, H1), lambda n: (0, 0)),
        ],
        out_specs=[
            pl.BlockSpec((bn, H1), lambda n: (n, 0)),
            pl.BlockSpec((H1, bn), lambda n: (0, n)),
        ],
        out_shape=[
            jax.ShapeDtypeStruct((N, H1), jnp.float32),
            jax.ShapeDtypeStruct((H1, N), jnp.float32),
        ],
    )(xwf, msgf, bias2, fc_w)


def _cmat_body(coefs_ref, b3_ref, ht_ref, out_ref):
    planes = []
    for r in range(R):
        q = coefs_ref[r, 0] * b3_ref[0] + coefs_ref[r, 1] * b3_ref[1]
        planes.append(jnp.dot(q, ht_ref[...],
                              preferred_element_type=jnp.float32))
    stacked = jnp.stack(planes, axis=-1)
    out_ref[...] = stacked.reshape(H1, stacked.shape[1] * R)


def _cmat(coefs, b3, ht):
    bi = 128
    return pl.pallas_call(
        _cmat_body,
        grid=(U // bi,),
        in_specs=[
            pl.BlockSpec(memory_space=pltpu.SMEM),
            pl.BlockSpec((2, H1, H1), lambda i: (0, 0, 0)),
            pl.BlockSpec((H1, bi), lambda i: (0, i + U // bi)),
        ],
        out_specs=pl.BlockSpec((H1, bi * R), lambda i: (0, i)),
        out_shape=jax.ShapeDtypeStruct((H1, U * R), jnp.float32),
    )(coefs, b3, ht)


def _dec_body(h_ref, c_ref, out_ref):
    out_ref[...] = jnp.dot(h_ref[...], c_ref[...],
                           preferred_element_type=jnp.float32)


def _dec(h, cmat):
    bu, bc = 256, 2048
    return pl.pallas_call(
        _dec_body,
        grid=(U // bu, U * R // bc),
        in_specs=[
            pl.BlockSpec((bu, H1), lambda u, j: (u, 0)),
            pl.BlockSpec((H1, bc), lambda u, j: (0, j)),
        ],
        out_specs=pl.BlockSpec((bu, bc), lambda u, j: (u, j)),
        out_shape=jax.ShapeDtypeStruct((U, U * R), jnp.float32),
    )(h, cmat)


def kernel(x, edge_index, edge_type, basis, comp, root, bias, fc_w,
           bidec_basis, coefs):
    bf = _projfc(basis.reshape(NB * N, H0), fc_w)
    rootf = _projfc(root, fc_w)
    wf = _wf(comp, bf.reshape(NB, N * H1), rootf.reshape(1, N * H1))
    xwf = _xwf(x, wf.reshape(R + 1, N, H1))
    cnt2 = _sck1(edge_index, edge_type)
    rcnt = _rcnt(cnt2.reshape(NC, R * N // 128, 128))
    msgf = _sck2(xwf.reshape((R + 1) * N, 128), edge_index, edge_type,
                 rcnt.reshape(R * N))
    h, ht = _relu(xwf, msgf.reshape(NC, N, H1), bias.reshape(1, H0), fc_w)
    b3 = bidec_basis.reshape(2, H1, H1)
    cmat = _cmat(coefs, b3, ht)
    out2 = _dec(h, cmat)
    return out2.reshape(U * (N - U), R)


# speedup vs baseline: 6.6203x; 6.6203x over previous
import functools

import jax
import jax.numpy as jnp
from jax import lax
from jax.experimental import pallas as pl
from jax.experimental.pallas import tpu as pltpu
from jax.experimental.pallas import tpu_sc as plsc

N = 4096
R = 5
NB = 30
H0 = 256
H1 = 64
E = 131072
U = 2048

NC, NS, L = 2, 16, 16
NT = NC * NS
EPT = E // NT


def _projfc_body(a_ref, fc_ref, out_ref):
    out_ref[...] = jnp.dot(a_ref[...], fc_ref[...],
                           preferred_element_type=jnp.float32)


def _projfc(a2d, fc_w):
    bm = 2048
    m = a2d.shape[0]
    return pl.pallas_call(
        _projfc_body,
        grid=(m // bm,),
        in_specs=[
            pl.BlockSpec((bm, H0), lambda i: (i, 0)),
            pl.BlockSpec((H0, H1), lambda i: (0, 0)),
        ],
        out_specs=pl.BlockSpec((bm, H1), lambda i: (i, 0)),
        out_shape=jax.ShapeDtypeStruct((m, H1), jnp.float32),
    )(a2d, fc_w)


def _wf_body(comp_ref, bf_ref, rootf_ref, out_ref):
    w = jnp.dot(comp_ref[...], bf_ref[...],
                preferred_element_type=jnp.float32)
    out_ref[...] = jnp.concatenate([w, rootf_ref[...]], axis=0)


def _wf(comp, bf_flat, rootf_flat):
    cb = 8192
    return pl.pallas_call(
        _wf_body,
        grid=(bf_flat.shape[1] // cb,),
        in_specs=[
            pl.BlockSpec((R, NB), lambda c: (0, 0)),
            pl.BlockSpec((NB, cb), lambda c: (0, c)),
            pl.BlockSpec((1, cb), lambda c: (0, c)),
        ],
        out_specs=pl.BlockSpec((R + 1, cb), lambda c: (0, c)),
        out_shape=jax.ShapeDtypeStruct((R + 1, N * H1), jnp.float32),
    )(comp, bf_flat, rootf_flat)


def _xwf_body(x_ref, w_ref, out_ref):
    k = pl.program_id(1)
    bn = x_ref.shape[0]
    for r in range(R + 1):
        p = jnp.dot(x_ref[...], w_ref[r], preferred_element_type=jnp.float32)
        p = jnp.concatenate([p, jnp.zeros((bn, 128 - H1), jnp.float32)],
                            axis=1)

        @pl.when(k == 0)
        def _():
            out_ref[r] = p

        @pl.when(k > 0)
        def _():
            out_ref[r] = out_ref[r] + p


def _xwf(x, w6):
    bn, bk = 1024, 512
    return pl.pallas_call(
        _xwf_body,
        grid=(N // bn, N // bk),
        in_specs=[
            pl.BlockSpec((bn, bk), lambda n, k: (n, k)),
            pl.BlockSpec((R + 1, bk, H1), lambda n, k: (0, k, 0)),
        ],
        out_specs=pl.BlockSpec((R + 1, bn, 128), lambda n, k: (0, n, 0)),
        out_shape=jax.ShapeDtypeStruct((R + 1, N, 128), jnp.float32),
    )(x, w6)


def _sck1(edge_index, edge_type):
    ka = 1024
    nchunks = EPT // ka
    zc = R * N // NS
    mesh = plsc.VectorSubcoreMesh(core_axis_name="c", subcore_axis_name="s")

    @functools.partial(
        pl.kernel,
        out_type=jax.ShapeDtypeStruct((NC, R * N), jnp.float32),
        mesh=mesh,
        scratch_types=[
            pltpu.VMEM_SHARED((R * N,), jnp.float32),
            pltpu.VMEM((ka,), jnp.int32),
            pltpu.VMEM((ka,), jnp.int32),
            pltpu.VMEM((ka,), jnp.int32),
            pltpu.VMEM((ka,), jnp.float32),
            pltpu.VMEM((zc,), jnp.float32),
        ],
    )
    def k(ei, et, cnt_out, cnt_s, dbuf, tbuf, cidx, ones, zbuf):
        c = lax.axis_index("c")
        s = lax.axis_index("s")
        wid = c * NS + s

        def zf(i, _):
            zbuf[pl.ds(i * L, L)] = jnp.zeros((L,), jnp.float32)
            return 0

        lax.fori_loop(0, zc // L, zf, 0)

        def of(i, _):
            ones[pl.ds(i * L, L)] = jnp.ones((L,), jnp.float32)
            return 0

        lax.fori_loop(0, ka // L, of, 0)

        pltpu.sync_copy(zbuf, cnt_s.at[pl.ds(s * zc, zc)])
        plsc.subcore_barrier()

        def chunk(i, _):
            base = wid * EPT + i * ka
            pltpu.sync_copy(ei.at[1, pl.ds(base, ka)], dbuf)
            pltpu.sync_copy(et.at[pl.ds(base, ka)], tbuf)

            def vec(j, _):
                d = dbuf[pl.ds(j * L, L)]
                t = tbuf[pl.ds(j * L, L)]
                cidx[pl.ds(j * L, L)] = t * N + d
                return 0

            lax.fori_loop(0, ka // L, vec, 0)
            pltpu.sync_copy(ones, cnt_s.at[cidx], add=True)
            return 0

        lax.fori_loop(0, nchunks, chunk, 0)
        plsc.subcore_barrier()
        pltpu.sync_copy(cnt_s.at[pl.ds(s * zc, zc)],
                        cnt_out.at[c, pl.ds(s * zc, zc)])

    return k(edge_index, edge_type)


def _rcnt_body(cnt_ref, out_ref):
    out_ref[...] = 1.0 / jnp.maximum(cnt_ref[0] + cnt_ref[1], 1.0)


def _rcnt(cnt2r):
    return pl.pallas_call(
        _rcnt_body,
        grid=(1,),
        in_specs=[pl.BlockSpec((NC, R * N // 128, 128), lambda i: (0, 0, 0))],
        out_specs=pl.BlockSpec((R * N // 128, 128), lambda i: (0, 0)),
        out_shape=jax.ShapeDtypeStruct((R * N // 128, 128), jnp.float32),
    )(cnt2r)


def _sck2(xwf_flat, edge_index, edge_type, rcnt_flat):
    kb = 128
    nchunks = EPT // kb
    spt = N * H1 // NS
    mesh = plsc.VectorSubcoreMesh(core_axis_name="c", subcore_axis_name="s")

    @functools.partial(
        pl.kernel,
        out_type=jax.ShapeDtypeStruct((NC * N * H1,), jnp.float32),
        mesh=mesh,
        scratch_types=[
            pltpu.VMEM_SHARED((N * H1,), jnp.float32),
            pltpu.VMEM((kb,), jnp.int32),
            pltpu.VMEM((kb,), jnp.int32),
            pltpu.VMEM((kb,), jnp.int32),
            pltpu.VMEM((kb,), jnp.int32),
            pltpu.VMEM((kb,), jnp.float32),
            pltpu.VMEM((kb, 128), jnp.float32),
            pltpu.VMEM((kb * H1,), jnp.float32),
            pltpu.VMEM((kb * H1,), jnp.int32),
            pltpu.VMEM((kb * H1,), jnp.float32),
            pltpu.VMEM((kb * H1,), jnp.int32),
            pltpu.VMEM((2048,), jnp.float32),
            pltpu.SemaphoreType.DMA,
            pltpu.SemaphoreType.DMA,
            pltpu.SemaphoreType.DMA,
            pltpu.SemaphoreType.DMA,
        ],
    )
    def k(xwf, ei, et, rcnt, msg_out, acc_s, sbuf, dbuf, tbuf, gidx, wbuf,
          rows, vals_a, eidx_a, vals_b, eidx_b, zbuf, sem, sem2, sem3, sem4):
        c = lax.axis_index("c")
        s = lax.axis_index("s")
        wid = c * NS + s

        def zf(i, _):
            zbuf[pl.ds(i * L, L)] = jnp.zeros((L,), jnp.float32)
            return 0

        lax.fori_loop(0, 2048 // L, zf, 0)
        for jj in range(spt // 2048):
            pltpu.sync_copy(zbuf, acc_s.at[pl.ds(s * spt + jj * 2048, 2048)])
        plsc.subcore_barrier()

        iotas = [lax.iota(jnp.int32, L) + h * L for h in range(H1 // L)]

        def subchunk(i, vals, eidx):
            base = wid * EPT + i * kb
            pltpu.sync_copy(ei.at[0, pl.ds(base, kb)], sbuf)
            pltpu.sync_copy(ei.at[1, pl.ds(base, kb)], dbuf)
            pltpu.sync_copy(et.at[pl.ds(base, kb)], tbuf)

            def vec(j, _):
                sv = sbuf[pl.ds(j * L, L)]
                dv = dbuf[pl.ds(j * L, L)]
                tv = tbuf[pl.ds(j * L, L)]
                gidx[pl.ds(j * L, L)] = tv * N + sv
                cidx = tv * N + dv
                dbuf[pl.ds(j * L, L)] = dv * H1
                tbuf[pl.ds(j * L, L)] = cidx
                return 0

            lax.fori_loop(0, kb // L, vec, 0)
            cp1 = pltpu.async_copy(xwf.at[gidx], rows, sem)
            cp2 = pltpu.async_copy(rcnt.at[tbuf], wbuf, sem2)
            cp1.wait()
            cp2.wait()

            def scale(j, _):
                wv = wbuf[pl.ds(j * L, L)]
                d64 = dbuf[pl.ds(j * L, L)]
                for e16 in range(L):
                    e = j * L + e16
                    w = wv[e16]
                    db = d64[e16]
                    for h in range(H1 // L):
                        vals[pl.ds(e * H1 + h * L, L)] = (
                            rows[e, pl.ds(h * L, L)] * w)
                        eidx[pl.ds(e * H1 + h * L, L)] = iotas[h] + db
                return 0

            lax.fori_loop(0, kb // L, scale, 0)

        def chunk(i, _):
            subchunk(2 * i, vals_a, eidx_a)
            cpa = pltpu.async_copy(vals_a, acc_s.at[eidx_a], sem3, add=True)
            subchunk(2 * i + 1, vals_b, eidx_b)
            cpa.wait()
            cpb = pltpu.async_copy(vals_b, acc_s.at[eidx_b], sem4, add=True)
            cpb.wait()
            return 0

        lax.fori_loop(0, nchunks // 2, chunk, 0)
        plsc.subcore_barrier()
        pltpu.sync_copy(acc_s.at[pl.ds(s * spt, spt)],
                        msg_out.at[pl.ds(c * N * H1 + s * spt, spt)])

    return k(xwf_flat, edge_index, edge_type, rcnt_flat)


def _relu_body(xwf_ref, msg_ref, bias_ref, fc_ref, h_ref, ht_ref):
    biasf = jnp.dot(bias_ref[...], fc_ref[...],
                    preferred_element_type=jnp.float32)
    h = jnp.maximum(xwf_ref[0, :, :H1] + msg_ref[0] + msg_ref[1] + biasf, 0.0)
    h_ref[...] = h
    ht_ref[...] = h.T


def _relu(xwf, msgf, bias2, fc_w):
    bn = 512
    return pl.pallas_call(
        _relu_body,
        grid=(N // bn,),
        in_specs=[
            pl.BlockSpec((1, bn, 128), lambda n: (R, n, 0)),
            pl.BlockSpec((NC, bn, H1), lambda n: (0, n, 0)),
            pl.BlockSpec((1, H0), lambda n: (0, 0)),
            pl.BlockSpec((H0, H1), lambda n: (0, 0)),
        ],
        out_specs=[
            pl.BlockSpec((bn, H1), lambda n: (n, 0)),
            pl.BlockSpec((H1, bn), lambda n: (0, n)),
        ],
        out_shape=[
            jax.ShapeDtypeStruct((N, H1), jnp.float32),
            jax.ShapeDtypeStruct((H1, N), jnp.float32),
        ],
    )(xwf, msgf, bias2, fc_w)


def _cmat_body(coefs_ref, b3_ref, ht_ref, out_ref):
    planes = []
    for r in range(R):
        q = coefs_ref[r, 0] * b3_ref[0] + coefs_ref[r, 1] * b3_ref[1]
        planes.append(jnp.dot(q, ht_ref[...],
                              preferred_element_type=jnp.float32))
    out_ref[...] = jnp.stack(planes, axis=0)


def _cmat(coefs, b3, ht):
    bi = 512
    return pl.pallas_call(
        _cmat_body,
        grid=(U // bi,),
        in_specs=[
            pl.BlockSpec(memory_space=pltpu.SMEM),
            pl.BlockSpec((2, H1, H1), lambda i: (0, 0, 0)),
            pl.BlockSpec((H1, bi), lambda i: (0, i + U // bi)),
        ],
        out_specs=pl.BlockSpec((R, H1, bi), lambda i: (0, 0, i)),
        out_shape=jax.ShapeDtypeStruct((R, H1, U), jnp.float32),
    )(coefs, b3, ht)


def _dec_body(h_ref, c_ref, out_ref):
    out_ref[0] = jnp.dot(h_ref[...], c_ref[0],
                         preferred_element_type=jnp.float32)


def _dec(h, c4):
    bu = 256
    ni = N - U
    return pl.pallas_call(
        _dec_body,
        grid=(R, U // bu),
        in_specs=[
            pl.BlockSpec((bu, H1), lambda r, u: (u, 0)),
            pl.BlockSpec((1, H1, ni), lambda r, u: (r, 0, 0)),
        ],
        out_specs=pl.BlockSpec((1, bu, ni), lambda r, u: (r, u, 0)),
        out_shape=jax.ShapeDtypeStruct((R, U, ni), jnp.float32),
    )(h, c4)


def kernel(x, edge_index, edge_type, basis, comp, root, bias, fc_w,
           bidec_basis, coefs):
    bf = _projfc(basis.reshape(NB * N, H0), fc_w)
    rootf = _projfc(root, fc_w)
    wf = _wf(comp, bf.reshape(NB, N * H1), rootf.reshape(1, N * H1))
    xwf = _xwf(x, wf.reshape(R + 1, N, H1))
    cnt2 = _sck1(edge_index, edge_type)
    rcnt = _rcnt(cnt2.reshape(NC, R * N // 128, 128))
    msgf = _sck2(xwf.reshape((R + 1) * N, 128), edge_index, edge_type,
                 rcnt.reshape(R * N))
    h, ht = _relu(xwf, msgf.reshape(NC, N, H1), bias.reshape(1, H0), fc_w)
    b3 = bidec_basis.reshape(2, H1, H1)
    c4 = _cmat(coefs, b3, ht)
    out5 = _dec(h, c4)
    return out5.reshape(R, U * (N - U)).T
